# baseline (speedup 1.0000x reference)
import functools

import jax
import jax.numpy as jnp
from jax import lax
from jax.experimental import pallas as pl
from jax.experimental.pallas import tpu as pltpu
from jax.experimental.pallas import tpu_sc as plsc

N = 4096
E = 131072
NF = 111
NF_PAD = 128
EMB = 32
HID = 64
NCLS = 15
NCLS_PAD = 128
HW_W = 80
EROWS = E // 128
NWORK = 32
ROWS_PER_W = EROWS // NWORK
NSLICE = N // 16
NBUF = 4

_f32 = jnp.float32


def _prep1_body(xp, wlin, blin, won, wco, hwon_o, hwco_o):
    h = jnp.dot(xp[...], wlin[...], preferred_element_type=_f32) + blin[...]
    ones = jnp.ones((N, 16), _f32)
    hwon_o[...] = jnp.concatenate(
        [jnp.dot(h, won[...], preferred_element_type=_f32), ones], axis=1)
    hwco_o[...] = jnp.concatenate(
        [jnp.dot(h, wco[...], preferred_element_type=_f32), ones], axis=1)


def _ei_body(ei, out):
    out[...] = ei[...].reshape(2, EROWS, 128)


def _marshal(ei_raw, ea_raw):
    ei = pl.pallas_call(
        _ei_body,
        out_shape=jax.ShapeDtypeStruct((2, EROWS, 128), jnp.int32),
    )(ei_raw)
    ea = ea_raw.reshape(EROWS, 512)
    return ei, ea


def _sc_body(hw, ei, ea1d, wa16, z80,
             agg_o,
             src_v, dst_v, attr_v, wa_v, r0, r1, r2, r3,
             agg_sp,
             gs0, gs1, gs2, gs3, ss0, ss1, ss2, ss3):
    cid = lax.axis_index("c")
    sid = lax.axis_index("s")
    wid = sid * 2 + cid

    zs = pl.ds(sid * NSLICE, NSLICE)
    pltpu.sync_copy(z80.at[zs], agg_sp.at[zs])
    plsc.subcore_barrier()

    wrow = wid * ROWS_PER_W
    bufs = (r0, r1, r2, r3)
    gs = (gs0, gs1, gs2, gs3)
    ss = (ss0, ss1, ss2, ss3)

    pltpu.sync_copy(ei.at[0, pl.ds(wrow, ROWS_PER_W)], src_v)
    pltpu.sync_copy(ei.at[1, pl.ds(wrow, ROWS_PER_W)], dst_v)
    pltpu.sync_copy(ea1d.at[pl.ds(wrow, ROWS_PER_W)], attr_v)
    pltpu.sync_copy(wa16, wa_v)
    wav = wa_v[...]
    iota4 = lax.iota(jnp.int32, 16) * 4

    for b in range(NBUF - 1):
        pltpu.async_copy(hw.at[src_v.at[b]], bufs[b], gs[b])

    def ring(k0, carry):
        for b in range(NBUF):
            k = k0 * NBUF + b
            rb = bufs[b]
            nb = (b + NBUF - 1) % NBUF
            pltpu.make_async_copy(hw.at[src_v.at[k]], rb, gs[b]).wait()

            @pl.when(k >= 1)
            def _():
                pltpu.make_async_copy(
                    bufs[nb], agg_sp.at[dst_v.at[k]], ss[nb]).wait()

            @pl.when(k + NBUF - 1 < ROWS_PER_W)
            def _():
                pltpu.async_copy(
                    hw.at[src_v.at[k + NBUF - 1]], bufs[nb], gs[nb])

            def mul(grp, c2):
                base = grp * 64
                krow = jnp.full((16,), k, jnp.int32)
                s = jnp.zeros((16,), _f32)
                for j in range(4):
                    aj = plsc.load_gather(attr_v, [krow, iota4 + (base + j)])
                    s = s + wav[j] * aj
                gvec = 1.0 / (1.0 + jnp.exp(-s))
                for t in range(16):
                    gb = gvec.at[jnp.full((16,), t, jnp.int32)].get(
                        mode="promise_in_bounds")
                    e = grp * 16 + t
                    for j in range(4):
                        sl = pl.ds(j * 16, 16)
                        rb[e, sl] = rb[e, sl] * gb
                return c2

            lax.fori_loop(0, 8, mul, 0)
            pltpu.async_copy(rb, agg_sp.at[dst_v.at[k]], ss[b], add=True)
        return carry

    lax.fori_loop(0, ROWS_PER_W // NBUF, ring, 0)
    pltpu.make_async_copy(bufs[NBUF - 1], agg_sp.at[dst_v.at[0]],
                          ss[NBUF - 1]).wait()

    plsc.subcore_barrier()
    pltpu.sync_copy(agg_sp.at[zs], agg_o.at[cid, zs])


def _combine_body(aon, aco, wsp, bsp, hid_o, spad_o):
    a1 = aon[0, :, :HID] + aon[1, :, :HID]
    a2 = aco[0, :, :HID] + aco[1, :, :HID]
    d1 = aon[0, :, HID:HID + 1] + aon[1, :, HID:HID + 1]
    d2 = aco[0, :, HID:HID + 1] + aco[1, :, HID:HID + 1]
    m1 = a1 / jnp.maximum(d1, 1.0)
    m2 = a2 / jnp.maximum(d2, 1.0)
    hid = jnp.maximum(m1 + m2, 0.0)
    hid_o[...] = hid
    logits = jnp.dot(hid, wsp[...], preferred_element_type=_f32) + bsp[...]
    spad_o[...] = jax.nn.softmax(logits, axis=-1)


def _pred_body(srow, sfull, gt, pred_o, loss_o):
    i = pl.program_id(0)
    p = lax.dot_general(srow[...], sfull[...], (((1,), (1,)), ((), ())),
                        preferred_element_type=_f32)
    pred_o[...] = p
    d = p - gt[...]
    part = jnp.sum(d * d)

    @pl.when(i == 0)
    def _():
        loss_o[0, 0] = 0.0

    loss_o[0, 0] += part


def kernel(x_note, edge_index_onset, edge_attr_onset, edge_index_consec,
           edge_attr_consec, grouping_matrix_true, W_lin, b_lin,
           W_onset, Wa_onset, W_consec, Wa_consec, W_s, b_s):
    xp = jnp.pad(x_note, ((0, 0), (0, NF_PAD - NF)))
    wlin = jnp.pad(W_lin, ((0, NF_PAD - NF), (0, 0)))
    blin = b_lin.reshape(1, EMB)

    hw_on, hw_co = pl.pallas_call(
        _prep1_body,
        out_shape=[
            jax.ShapeDtypeStruct((N, HW_W), _f32),
            jax.ShapeDtypeStruct((N, HW_W), _f32),
        ],
    )(xp, wlin, blin, W_onset, W_consec)

    ei_on, ea_on = _marshal(edge_index_onset, edge_attr_onset)
    ei_co, ea_co = _marshal(edge_index_consec, edge_attr_consec)
    wa_on = jnp.pad(Wa_onset[:, 0], (0, 12))
    wa_co = jnp.pad(Wa_consec[:, 0], (0, 12))
    z80 = jnp.zeros((N, HW_W), _f32)

    mesh = plsc.VectorSubcoreMesh(core_axis_name="c", subcore_axis_name="s")
    mp = pl.kernel(
        _sc_body,
        out_type=jax.ShapeDtypeStruct((2, N, HW_W), _f32),
        mesh=mesh,
        compiler_params=pltpu.CompilerParams(use_tc_tiling_on_sc=False,
                                             needs_layout_passes=False),
        scratch_types=[
            pltpu.VMEM((ROWS_PER_W, 128), jnp.int32),
            pltpu.VMEM((ROWS_PER_W, 128), jnp.int32),
            pltpu.VMEM((ROWS_PER_W, 512), _f32),
            pltpu.VMEM((16,), _f32),
            pltpu.VMEM((128, HW_W), _f32),
            pltpu.VMEM((128, HW_W), _f32),
            pltpu.VMEM((128, HW_W), _f32),
            pltpu.VMEM((128, HW_W), _f32),
            pltpu.VMEM_SHARED((N, HW_W), _f32),
        ] + [pltpu.SemaphoreType.DMA] * 8,
    )
    agg_on = mp(hw_on, ei_on, ea_on, wa_on, z80)
    agg_co = mp(hw_co, ei_co, ea_co, wa_co, z80)

    wsp = jnp.pad(W_s, ((0, 0), (0, NCLS_PAD - NCLS)))
    bsp = jnp.pad(b_s, (0, NCLS_PAD - NCLS),
                  constant_values=-1e30).reshape(1, NCLS_PAD)

    hidden, spad = pl.pallas_call(
        _combine_body,
        out_shape=[
            jax.ShapeDtypeStruct((N, HID), _f32),
            jax.ShapeDtypeStruct((N, NCLS_PAD), _f32),
        ],
    )(agg_on, agg_co, wsp, bsp)

    nblk = 32
    pred, loss_sum = pl.pallas_call(
        _pred_body,
        grid=(nblk,),
        in_specs=[
            pl.BlockSpec((N // nblk, NCLS_PAD), lambda i: (i, 0)),
            pl.BlockSpec((N, NCLS_PAD), lambda i: (0, 0)),
            pl.BlockSpec((N // nblk, N), lambda i: (i, 0)),
        ],
        out_specs=[
            pl.BlockSpec((N // nblk, N), lambda i: (i, 0)),
            pl.BlockSpec(memory_space=pltpu.SMEM),
        ],
        out_shape=[
            jax.ShapeDtypeStruct((N, N), _f32),
            jax.ShapeDtypeStruct((1, 1), _f32),
        ],
    )(spad, spad, grouping_matrix_true)

    S = spad[:, :NCLS]
    loss = loss_sum[0, 0] / float(N * N)
    return hidden, S, loss, pred


# speedup vs baseline: 1.1460x; 1.1460x over previous
import functools

import jax
import jax.numpy as jnp
from jax import lax
from jax.experimental import pallas as pl
from jax.experimental.pallas import tpu as pltpu
from jax.experimental.pallas import tpu_sc as plsc

N = 4096
E = 131072
NF = 111
NF_PAD = 128
EMB = 32
HID = 64
NCLS = 15
NCLS_PAD = 128
HW_W = 80
EROWS = E // 128
NWORK = 32
ROWS_PER_W = EROWS // NWORK
NSLICE = N // 16
NBUF = 4

_f32 = jnp.float32


def _prep1_body(xp, wlin, blin, won, wco, hwon_o, hwco_o):
    h = jnp.dot(xp[...], wlin[...], preferred_element_type=_f32) + blin[...]
    ones = jnp.ones((N, 16), _f32)
    hwon_o[...] = jnp.concatenate(
        [jnp.dot(h, won[...], preferred_element_type=_f32), ones], axis=1)
    hwco_o[...] = jnp.concatenate(
        [jnp.dot(h, wco[...], preferred_element_type=_f32), ones], axis=1)


def _ea_t_body(ea, out):
    out[...] = ea[...].T


def _marshal(ei_raw, ea_raw):
    ei = ei_raw.reshape(2, EROWS, 128)
    ea_t = pl.pallas_call(
        _ea_t_body,
        grid=(16,),
        in_specs=[pl.BlockSpec((E // 16, 4), lambda i: (i, 0))],
        out_specs=pl.BlockSpec((4, E // 16), lambda i: (0, i)),
        out_shape=jax.ShapeDtypeStruct((4, E), _f32),
    )(ea_raw)
    return ei, ea_t.reshape(4, EROWS, 128)


def _sc_body(hw, ei, ea1d, wa16, z80,
             agg_o,
             src_v, dst_v, attr_v, wa_v, r0, r1, r2, r3,
             agg_sp,
             gs0, gs1, gs2, gs3, ss0, ss1, ss2, ss3):
    cid = lax.axis_index("c")
    sid = lax.axis_index("s")
    wid = sid * 2 + cid

    zs = pl.ds(sid * NSLICE, NSLICE)
    pltpu.sync_copy(z80.at[zs], agg_sp.at[zs])
    plsc.subcore_barrier()

    wrow = wid * ROWS_PER_W
    bufs = (r0, r1, r2, r3)
    gs = (gs0, gs1, gs2, gs3)
    ss = (ss0, ss1, ss2, ss3)

    pltpu.sync_copy(ei.at[0, pl.ds(wrow, ROWS_PER_W)], src_v)
    pltpu.sync_copy(ei.at[1, pl.ds(wrow, ROWS_PER_W)], dst_v)
    for j in range(4):
        pltpu.sync_copy(ea1d.at[j, pl.ds(wrow, ROWS_PER_W)], attr_v.at[j])
    pltpu.sync_copy(wa16, wa_v)
    wav = wa_v[...]

    for b in range(NBUF - 1):
        pltpu.async_copy(hw.at[src_v.at[b]], bufs[b], gs[b])

    def ring(k0, carry):
        for b in range(NBUF):
            k = k0 * NBUF + b
            rb = bufs[b]
            nb = (b + NBUF - 1) % NBUF
            pltpu.make_async_copy(hw.at[src_v.at[k]], rb, gs[b]).wait()

            @pl.when(k >= 1)
            def _():
                pltpu.make_async_copy(
                    bufs[nb], agg_sp.at[dst_v.at[k]], ss[nb]).wait()

            @pl.when(k + NBUF - 1 < ROWS_PER_W)
            def _():
                pltpu.async_copy(
                    hw.at[src_v.at[k + NBUF - 1]], bufs[nb], gs[nb])

            def mul(grp, c2):
                sl16 = pl.ds(grp * 16, 16)
                s = jnp.zeros((16,), _f32)
                for j in range(4):
                    aj = attr_v[j, k, sl16]
                    s = s + wav[j] * aj
                gvec = 1.0 / (1.0 + jnp.exp(-s))
                for t in range(16):
                    gb = gvec.at[jnp.full((16,), t, jnp.int32)].get(
                        mode="promise_in_bounds")
                    e = grp * 16 + t
                    for j in range(4):
                        sl = pl.ds(j * 16, 16)
                        rb[e, sl] = rb[e, sl] * gb
                return c2

            lax.fori_loop(0, 8, mul, 0)
            pltpu.async_copy(rb, agg_sp.at[dst_v.at[k]], ss[b], add=True)
        return carry

    lax.fori_loop(0, ROWS_PER_W // NBUF, ring, 0)
    pltpu.make_async_copy(bufs[NBUF - 1], agg_sp.at[dst_v.at[0]],
                          ss[NBUF - 1]).wait()

    plsc.subcore_barrier()
    pltpu.sync_copy(agg_sp.at[zs], agg_o.at[cid, zs])


def _combine_body(aon, aco, wsp, bsp, hid_o, spad_o):
    a1 = aon[0, :, :HID] + aon[1, :, :HID]
    a2 = aco[0, :, :HID] + aco[1, :, :HID]
    d1 = aon[0, :, HID:HID + 1] + aon[1, :, HID:HID + 1]
    d2 = aco[0, :, HID:HID + 1] + aco[1, :, HID:HID + 1]
    m1 = a1 / jnp.maximum(d1, 1.0)
    m2 = a2 / jnp.maximum(d2, 1.0)
    hid = jnp.maximum(m1 + m2, 0.0)
    hid_o[...] = hid
    logits = jnp.dot(hid, wsp[...], preferred_element_type=_f32) + bsp[...]
    spad_o[...] = jax.nn.softmax(logits, axis=-1)


def _pred_body(srow, sfull, gt, pred_o, loss_o):
    i = pl.program_id(0)
    p = lax.dot_general(srow[...], sfull[...], (((1,), (1,)), ((), ())),
                        preferred_element_type=_f32)
    pred_o[...] = p
    d = p - gt[...]
    part = jnp.sum(d * d)

    @pl.when(i == 0)
    def _():
        loss_o[0, 0] = 0.0

    loss_o[0, 0] += part


def kernel(x_note, edge_index_onset, edge_attr_onset, edge_index_consec,
           edge_attr_consec, grouping_matrix_true, W_lin, b_lin,
           W_onset, Wa_onset, W_consec, Wa_consec, W_s, b_s):
    xp = jnp.pad(x_note, ((0, 0), (0, NF_PAD - NF)))
    wlin = jnp.pad(W_lin, ((0, NF_PAD - NF), (0, 0)))
    blin = b_lin.reshape(1, EMB)

    hw_on, hw_co = pl.pallas_call(
        _prep1_body,
        out_shape=[
            jax.ShapeDtypeStruct((N, HW_W), _f32),
            jax.ShapeDtypeStruct((N, HW_W), _f32),
        ],
    )(xp, wlin, blin, W_onset, W_consec)

    ei_on, ea_on = _marshal(edge_index_onset, edge_attr_onset)
    ei_co, ea_co = _marshal(edge_index_consec, edge_attr_consec)
    wa_on = jnp.pad(Wa_onset[:, 0], (0, 12))
    wa_co = jnp.pad(Wa_consec[:, 0], (0, 12))
    z80 = jnp.zeros((N, HW_W), _f32)

    mesh = plsc.VectorSubcoreMesh(core_axis_name="c", subcore_axis_name="s")
    mp = pl.kernel(
        _sc_body,
        out_type=jax.ShapeDtypeStruct((2, N, HW_W), _f32),
        mesh=mesh,
        compiler_params=pltpu.CompilerParams(use_tc_tiling_on_sc=False,
                                             needs_layout_passes=False),
        scratch_types=[
            pltpu.VMEM((ROWS_PER_W, 128), jnp.int32),
            pltpu.VMEM((ROWS_PER_W, 128), jnp.int32),
            pltpu.VMEM((4, ROWS_PER_W, 128), _f32),
            pltpu.VMEM((16,), _f32),
            pltpu.VMEM((128, HW_W), _f32),
            pltpu.VMEM((128, HW_W), _f32),
            pltpu.VMEM((128, HW_W), _f32),
            pltpu.VMEM((128, HW_W), _f32),
            pltpu.VMEM_SHARED((N, HW_W), _f32),
        ] + [pltpu.SemaphoreType.DMA] * 8,
    )
    agg_on = mp(hw_on, ei_on, ea_on, wa_on, z80)
    agg_co = mp(hw_co, ei_co, ea_co, wa_co, z80)

    wsp = jnp.pad(W_s, ((0, 0), (0, NCLS_PAD - NCLS)))
    bsp = jnp.pad(b_s, (0, NCLS_PAD - NCLS),
                  constant_values=-1e30).reshape(1, NCLS_PAD)

    hidden, spad = pl.pallas_call(
        _combine_body,
        out_shape=[
            jax.ShapeDtypeStruct((N, HID), _f32),
            jax.ShapeDtypeStruct((N, NCLS_PAD), _f32),
        ],
    )(agg_on, agg_co, wsp, bsp)

    nblk = 32
    pred, loss_sum = pl.pallas_call(
        _pred_body,
        grid=(nblk,),
        in_specs=[
            pl.BlockSpec((N // nblk, NCLS_PAD), lambda i: (i, 0)),
            pl.BlockSpec((N, NCLS_PAD), lambda i: (0, 0)),
            pl.BlockSpec((N // nblk, N), lambda i: (i, 0)),
        ],
        out_specs=[
            pl.BlockSpec((N // nblk, N), lambda i: (i, 0)),
            pl.BlockSpec(memory_space=pltpu.SMEM),
        ],
        out_shape=[
            jax.ShapeDtypeStruct((N, N), _f32),
            jax.ShapeDtypeStruct((1, 1), _f32),
        ],
    )(spad, spad, grouping_matrix_true)

    S = spad[:, :NCLS]
    loss = loss_sum[0, 0] / float(N * N)
    return hidden, S, loss, pred


# speedup vs baseline: 1.1478x; 1.0016x over previous
import jax
import jax.numpy as jnp
from jax import lax
from jax.experimental import pallas as pl
from jax.experimental.pallas import tpu as pltpu
from jax.experimental.pallas import tpu_sc as plsc

N = 4096
E = 131072
NF = 111
NF_PAD = 128
EMB = 32
HID = 64
NCLS = 15
NCLS_PAD = 128
HW_W = 80
EROWS = E // 128
NWORK = 32
ROWS_PER_W = EROWS // NWORK
NSLICE = N // 16
NBUF = 4

_f32 = jnp.float32


def _prep1_body(xp, wlin, blin, won, wco, hwon_o, hwco_o):
    h = jnp.dot(xp[...], wlin[...], preferred_element_type=_f32) + blin[...]
    ones = jnp.ones((N, 16), _f32)
    hwon_o[...] = jnp.concatenate(
        [jnp.dot(h, won[...], preferred_element_type=_f32), ones], axis=1)
    hwco_o[...] = jnp.concatenate(
        [jnp.dot(h, wco[...], preferred_element_type=_f32), ones], axis=1)


def _ea_t_body(ea, out):
    out[...] = ea[...].T


def _marshal(ei_raw, ea_raw):
    ei = ei_raw.reshape(2, EROWS, 128)
    ea_t = pl.pallas_call(
        _ea_t_body,
        grid=(16,),
        in_specs=[pl.BlockSpec((E // 16, 4), lambda i: (i, 0))],
        out_specs=pl.BlockSpec((4, E // 16), lambda i: (0, i)),
        out_shape=jax.ShapeDtypeStruct((4, E), _f32),
    )(ea_raw)
    return ei, ea_t.reshape(4, EROWS, 128)


def _sc_body(hw, ei, ea1d, wa16, z80,
             agg_o,
             src_v, dst_v, attr_v, wa_v, r0, r1, r2, r3,
             agg_sp,
             gs0, gs1, gs2, gs3, ss0, ss1, ss2, ss3):
    cid = lax.axis_index("c")
    sid = lax.axis_index("s")
    wid = sid * 2 + cid

    zs = pl.ds(sid * NSLICE, NSLICE)
    pltpu.sync_copy(z80.at[zs], agg_sp.at[zs])
    plsc.subcore_barrier()

    wrow = wid * ROWS_PER_W
    bufs = (r0, r1, r2, r3)
    gs = (gs0, gs1, gs2, gs3)
    ss = (ss0, ss1, ss2, ss3)

    pltpu.sync_copy(ei.at[0, pl.ds(wrow, ROWS_PER_W)], src_v)
    pltpu.sync_copy(ei.at[1, pl.ds(wrow, ROWS_PER_W)], dst_v)
    for j in range(4):
        pltpu.sync_copy(ea1d.at[j, pl.ds(wrow, ROWS_PER_W)], attr_v.at[j])
    pltpu.sync_copy(wa16, wa_v)
    wav = wa_v[...]

    for b in range(NBUF - 1):
        pltpu.async_copy(hw.at[src_v.at[b]], bufs[b], gs[b])

    def ring(k0, carry):
        for b in range(NBUF):
            k = k0 * NBUF + b
            rb = bufs[b]
            nb = (b + NBUF - 1) % NBUF
            pltpu.make_async_copy(hw.at[src_v.at[k]], rb, gs[b]).wait()

            @pl.when(k >= 1)
            def _():
                pltpu.make_async_copy(
                    bufs[nb], agg_sp.at[dst_v.at[k]], ss[nb]).wait()

            @pl.when(k + NBUF - 1 < ROWS_PER_W)
            def _():
                pltpu.async_copy(
                    hw.at[src_v.at[k + NBUF - 1]], bufs[nb], gs[nb])

            def mul(grp, c2):
                sl16 = pl.ds(grp * 16, 16)
                s = jnp.zeros((16,), _f32)
                for j in range(4):
                    aj = attr_v[j, k, sl16]
                    s = s + wav[j] * aj
                gvec = 1.0 / (1.0 + jnp.exp(-s))
                for t in range(16):
                    gb = gvec.at[jnp.full((16,), t, jnp.int32)].get(
                        mode="promise_in_bounds")
                    e = grp * 16 + t
                    for j in range(4):
                        sl = pl.ds(j * 16, 16)
                        rb[e, sl] = rb[e, sl] * gb
                return c2

            lax.fori_loop(0, 8, mul, 0)
            pltpu.async_copy(rb, agg_sp.at[dst_v.at[k]], ss[b], add=True)
        return carry

    lax.fori_loop(0, ROWS_PER_W // NBUF, ring, 0)
    pltpu.make_async_copy(bufs[NBUF - 1], agg_sp.at[dst_v.at[0]],
                          ss[NBUF - 1]).wait()

    plsc.subcore_barrier()
    pltpu.sync_copy(agg_sp.at[zs], agg_o.at[cid, zs])


def _combine_body(aon, aco, wsp, bsp, hid_o, spad_o):
    a1 = aon[0, :, :HID] + aon[1, :, :HID]
    a2 = aco[0, :, :HID] + aco[1, :, :HID]
    d1 = aon[0, :, HID:HID + 1] + aon[1, :, HID:HID + 1]
    d2 = aco[0, :, HID:HID + 1] + aco[1, :, HID:HID + 1]
    m1 = a1 / jnp.maximum(d1, 1.0)
    m2 = a2 / jnp.maximum(d2, 1.0)
    hid = jnp.maximum(m1 + m2, 0.0)
    hid_o[...] = hid
    logits = jnp.dot(hid, wsp[...], preferred_element_type=_f32) + bsp[...]
    spad_o[...] = jax.nn.softmax(logits, axis=-1)


def _pred_body(srow, sfull, gt, pred_o, loss_o):
    i = pl.program_id(0)
    p = lax.dot_general(srow[...], sfull[...], (((1,), (1,)), ((), ())),
                        preferred_element_type=_f32)
    pred_o[...] = p
    d = p - gt[...]
    part = jnp.sum(d * d)

    @pl.when(i == 0)
    def _():
        loss_o[0, 0] = 0.0

    loss_o[0, 0] += part


def kernel(x_note, edge_index_onset, edge_attr_onset, edge_index_consec,
           edge_attr_consec, grouping_matrix_true, W_lin, b_lin,
           W_onset, Wa_onset, W_consec, Wa_consec, W_s, b_s):
    xp = jnp.pad(x_note, ((0, 0), (0, NF_PAD - NF)))
    wlin = jnp.pad(W_lin, ((0, NF_PAD - NF), (0, 0)))
    blin = b_lin.reshape(1, EMB)

    hw_on, hw_co = pl.pallas_call(
        _prep1_body,
        out_shape=[
            jax.ShapeDtypeStruct((N, HW_W), _f32),
            jax.ShapeDtypeStruct((N, HW_W), _f32),
        ],
    )(xp, wlin, blin, W_onset, W_consec)

    ei_on, ea_on = _marshal(edge_index_onset, edge_attr_onset)
    ei_co, ea_co = _marshal(edge_index_consec, edge_attr_consec)
    wa_on = jnp.pad(Wa_onset[:, 0], (0, 12))
    wa_co = jnp.pad(Wa_consec[:, 0], (0, 12))
    z80 = jnp.zeros((N, HW_W), _f32)

    mesh = plsc.VectorSubcoreMesh(core_axis_name="c", subcore_axis_name="s")
    mp = pl.kernel(
        _sc_body,
        out_type=jax.ShapeDtypeStruct((2, N, HW_W), _f32),
        mesh=mesh,
        compiler_params=pltpu.CompilerParams(use_tc_tiling_on_sc=False,
                                             needs_layout_passes=False),
        scratch_types=[
            pltpu.VMEM((ROWS_PER_W, 128), jnp.int32),
            pltpu.VMEM((ROWS_PER_W, 128), jnp.int32),
            pltpu.VMEM((4, ROWS_PER_W, 128), _f32),
            pltpu.VMEM((16,), _f32),
            pltpu.VMEM((128, HW_W), _f32),
            pltpu.VMEM((128, HW_W), _f32),
            pltpu.VMEM((128, HW_W), _f32),
            pltpu.VMEM((128, HW_W), _f32),
            pltpu.VMEM_SHARED((N, HW_W), _f32),
        ] + [pltpu.SemaphoreType.DMA] * 8,
    )
    agg_on = mp(hw_on, ei_on, ea_on, wa_on, z80)
    agg_co = mp(hw_co, ei_co, ea_co, wa_co, z80)

    wsp = jnp.pad(W_s, ((0, 0), (0, NCLS_PAD - NCLS)))
    bsp = jnp.pad(b_s, (0, NCLS_PAD - NCLS),
                  constant_values=-1e30).reshape(1, NCLS_PAD)

    hidden, spad = pl.pallas_call(
        _combine_body,
        out_shape=[
            jax.ShapeDtypeStruct((N, HID), _f32),
            jax.ShapeDtypeStruct((N, NCLS_PAD), _f32),
        ],
    )(agg_on, agg_co, wsp, bsp)

    nblk = 32
    pred, loss_sum = pl.pallas_call(
        _pred_body,
        grid=(nblk,),
        in_specs=[
            pl.BlockSpec((N // nblk, NCLS_PAD), lambda i: (i, 0)),
            pl.BlockSpec((N, NCLS_PAD), lambda i: (0, 0)),
            pl.BlockSpec((N // nblk, N), lambda i: (i, 0)),
        ],
        out_specs=[
            pl.BlockSpec((N // nblk, N), lambda i: (i, 0)),
            pl.BlockSpec(memory_space=pltpu.SMEM),
        ],
        out_shape=[
            jax.ShapeDtypeStruct((N, N), _f32),
            jax.ShapeDtypeStruct((1, 1), _f32),
        ],
    )(spad, spad, grouping_matrix_true)

    S = spad[:, :NCLS]
    loss = loss_sum[0, 0] / float(N * N)
    return hidden, S, loss, pred
